# Initial kernel scaffold; baseline (speedup 1.0000x reference)
import jax
import jax.numpy as jnp
from jax.experimental import pallas as pl


def kernel(input, indices):
    raise NotImplementedError("write your pallas kernel here")


# speedup vs baseline: 4.0423x; 4.0423x over previous
import jax
import jax.numpy as jnp
from jax import lax
from jax.experimental import pallas as pl
from jax.experimental.pallas import tpu as pltpu
from jax.experimental.pallas import tpu_sc as plsc

B = 64
S_IN = 50176
S_OUT = 401408
TOTAL = B * S_IN
OUT_TOTAL = B * S_OUT
NW = 32
SEG = 100352
NSEG = OUT_TOTAL // SEG
SEG_PER_W = NSEG // NW
CHUNK = 6272
KBUF = CHUNK + 16
SENTINEL = 0x7F000000
L = 16
ST_LEN = 264


def _scatter_body(sk_hbm, sv_hbm, st_hbm, out_hbm, key_v, val_v, st_v, seg_v):
    wid = lax.axis_index("s") * 2 + lax.axis_index("c")
    pltpu.sync_copy(st_hbm.at[pl.ds(wid * SEG_PER_W, L)], st_v)
    lanes = lax.iota(jnp.int32, L)
    sts = st_v[...]

    for g in range(SEG_PER_W):
        base = (wid * SEG_PER_W + g) * SEG
        a0 = jnp.sum(jnp.where(lanes == g, sts, 0))
        a1 = jnp.sum(jnp.where(lanes == g + 1, sts, 0))
        off0 = (a0 // 8) * 8
        nch = (a1 - off0 + CHUNK - 1) // CHUNK

        def zb(i, c):
            seg_v[pl.ds(i * L, L)] = jnp.zeros((L,), jnp.float32)
            return c

        lax.fori_loop(0, SEG // L, zb, 0)

        def cb(j, c):
            off = off0 + j * CHUNK
            pltpu.sync_copy(sk_hbm.at[pl.ds(off, KBUF)], key_v)
            pltpu.sync_copy(sv_hbm.at[pl.ds(off, CHUNK)], val_v)

            def vb(i, c2):
                k = key_v[pl.ds(i * L, L)]
                kn = key_v[pl.ds(i * L + 1, L)]
                v = val_v[pl.ds(i * L, L)]
                loc = k - base
                m = (loc >= 0) & (loc < SEG) & (k != kn)
                plsc.store_scatter(seg_v, [jnp.where(m, loc, 0)], v, mask=m)
                return c2

            lax.fori_loop(0, CHUNK // L, vb, 0)
            return c

        lax.fori_loop(0, nch, cb, 0)
        pltpu.sync_copy(seg_v, out_hbm.at[pl.ds(base, SEG)])


def kernel(input, indices):
    idxf = indices.reshape(B, S_IN)
    rows = jnp.arange(B, dtype=jnp.int32)[:, None]
    gkey = (rows * S_OUT + idxf).reshape(-1)
    vals = input.reshape(-1)
    sk, sv = lax.sort((gkey, vals), dimension=0, is_stable=False, num_keys=1)
    skp = jnp.concatenate([sk, jnp.full((KBUF,), SENTINEL, jnp.int32)])
    svp = jnp.concatenate([sv, jnp.zeros((KBUF,), jnp.float32)])
    bounds = jnp.arange(NSEG + 1, dtype=jnp.int32) * SEG
    starts = jnp.searchsorted(sk, bounds, side="left").astype(jnp.int32)
    starts = jnp.concatenate(
        [starts, jnp.full((ST_LEN - NSEG - 1,), TOTAL, jnp.int32)])

    mesh = plsc.VectorSubcoreMesh(core_axis_name="c", subcore_axis_name="s")
    out = pl.kernel(
        _scatter_body,
        out_type=jax.ShapeDtypeStruct((OUT_TOTAL,), jnp.float32),
        scratch_types=[
            pltpu.VMEM((KBUF,), jnp.int32),
            pltpu.VMEM((CHUNK,), jnp.float32),
            pltpu.VMEM((L,), jnp.int32),
            pltpu.VMEM((SEG,), jnp.float32),
        ],
        mesh=mesh,
        compiler_params=pltpu.CompilerParams(needs_layout_passes=False),
    )(skp, svp, starts)
    return out.reshape(2, 32, 32, 112, 112)


# speedup vs baseline: 4.2639x; 1.0548x over previous
import jax
import jax.numpy as jnp
from jax import lax
from jax.experimental import pallas as pl
from jax.experimental.pallas import tpu as pltpu
from jax.experimental.pallas import tpu_sc as plsc

B = 64
S_IN = 50176
S_OUT = 401408
TOTAL = B * S_IN
OUT_TOTAL = B * S_OUT
NW = 32
SEG = 100352
NSEG = OUT_TOTAL // SEG
SEG_PER_W = NSEG // NW
CHUNK = 12544
KBUF = CHUNK + 16
UNROLL = 4
ZUNROLL = 8
SENTINEL = 0x7F000000
L = 16
ST_LEN = 264


def _scatter_body(sk_hbm, sv_hbm, st_hbm, out_hbm, key_v, val_v, st_v, seg_v):
    wid = lax.axis_index("s") * 2 + lax.axis_index("c")
    pltpu.sync_copy(st_hbm.at[pl.ds(wid * SEG_PER_W, L)], st_v)
    lanes = lax.iota(jnp.int32, L)
    sts = st_v[...]

    for g in range(SEG_PER_W):
        base = (wid * SEG_PER_W + g) * SEG
        a0 = jnp.sum(jnp.where(lanes == g, sts, 0))
        a1 = jnp.sum(jnp.where(lanes == g + 1, sts, 0))
        off0 = (a0 // 8) * 8
        nch = (a1 - off0 + CHUNK - 1) // CHUNK

        def zb(i, c):
            for u in range(ZUNROLL):
                seg_v[pl.ds((i * ZUNROLL + u) * L, L)] = jnp.zeros(
                    (L,), jnp.float32)
            return c

        lax.fori_loop(0, SEG // (L * ZUNROLL), zb, 0)

        def cb(j, c):
            off = off0 + j * CHUNK
            pltpu.sync_copy(sk_hbm.at[pl.ds(off, KBUF)], key_v)
            pltpu.sync_copy(sv_hbm.at[pl.ds(off, CHUNK)], val_v)

            def vb(i, c2):
                for u in range(UNROLL):
                    b = (i * UNROLL + u) * L
                    k = key_v[pl.ds(b, L)]
                    kn = key_v[pl.ds(b + 1, L)]
                    v = val_v[pl.ds(b, L)]
                    loc = k - base
                    m = (loc >= 0) & (loc < SEG) & (k != kn)
                    plsc.store_scatter(seg_v, [jnp.where(m, loc, 0)], v,
                                       mask=m)
                return c2

            lax.fori_loop(0, CHUNK // (L * UNROLL), vb, 0)
            return c

        lax.fori_loop(0, nch, cb, 0)
        pltpu.sync_copy(seg_v, out_hbm.at[pl.ds(base, SEG)])


def kernel(input, indices):
    idxf = indices.reshape(B, S_IN)
    rows = jnp.arange(B, dtype=jnp.int32)[:, None]
    gkey = (rows * S_OUT + idxf).reshape(-1)
    vals = input.reshape(-1)
    sk, sv = lax.sort((gkey, vals), dimension=0, is_stable=False, num_keys=1)
    skp = jnp.concatenate([sk, jnp.full((KBUF,), SENTINEL, jnp.int32)])
    svp = jnp.concatenate([sv, jnp.zeros((KBUF,), jnp.float32)])
    bounds = jnp.arange(NSEG + 1, dtype=jnp.int32) * SEG
    starts = jnp.searchsorted(sk, bounds, side="left").astype(jnp.int32)
    starts = jnp.concatenate(
        [starts, jnp.full((ST_LEN - NSEG - 1,), TOTAL, jnp.int32)])

    mesh = plsc.VectorSubcoreMesh(core_axis_name="c", subcore_axis_name="s")
    out = pl.kernel(
        _scatter_body,
        out_type=jax.ShapeDtypeStruct((OUT_TOTAL,), jnp.float32),
        scratch_types=[
            pltpu.VMEM((KBUF,), jnp.int32),
            pltpu.VMEM((CHUNK,), jnp.float32),
            pltpu.VMEM((L,), jnp.int32),
            pltpu.VMEM((SEG,), jnp.float32),
        ],
        mesh=mesh,
        compiler_params=pltpu.CompilerParams(needs_layout_passes=False),
    )(skp, svp, starts)
    return out.reshape(2, 32, 32, 112, 112)


# speedup vs baseline: 4.2656x; 1.0004x over previous
import jax
import jax.numpy as jnp
from jax import lax
from jax.experimental import pallas as pl
from jax.experimental.pallas import tpu as pltpu
from jax.experimental.pallas import tpu_sc as plsc

B = 64
S_IN = 50176
S_OUT = 401408
TOTAL = B * S_IN
OUT_TOTAL = B * S_OUT
NW = 32
SEG = 100352
NSEG = OUT_TOTAL // SEG
SEG_PER_W = NSEG // NW
CHUNK = 12544
KBUF = CHUNK + 16
UNROLL = 4
ZUNROLL = 8
SENTINEL = 0x7F000000
L = 16
ST_LEN = 264


def _scatter_body(sk_hbm, sv_hbm, st_hbm, out_hbm, key_v, val_v, st_v, seg_v):
    wid = lax.axis_index("s") * 2 + lax.axis_index("c")
    pltpu.sync_copy(st_hbm.at[pl.ds(wid * SEG_PER_W, L)], st_v)
    lanes = lax.iota(jnp.int32, L)
    sts = st_v[...]

    for g in range(SEG_PER_W):
        base = (wid * SEG_PER_W + g) * SEG
        a0 = jnp.sum(jnp.where(lanes == g, sts, 0))
        a1 = jnp.sum(jnp.where(lanes == g + 1, sts, 0))
        off0 = (a0 // 8) * 8
        nch = (a1 - off0 + CHUNK - 1) // CHUNK

        def zb(i, c):
            for u in range(ZUNROLL):
                seg_v[pl.ds((i * ZUNROLL + u) * L, L)] = jnp.zeros(
                    (L,), jnp.float32)
            return c

        lax.fori_loop(0, SEG // (L * ZUNROLL), zb, 0)

        def cb(j, c):
            off = off0 + j * CHUNK
            pltpu.sync_copy(sk_hbm.at[pl.ds(off, KBUF)], key_v)
            pltpu.sync_copy(sv_hbm.at[pl.ds(off, CHUNK)], val_v)

            def vb(i, c2):
                for u in range(UNROLL):
                    b = (i * UNROLL + u) * L
                    k = key_v[pl.ds(b, L)]
                    kn = key_v[pl.ds(b + 1, L)]
                    v = val_v[pl.ds(b, L)]
                    loc = k - base
                    m = (loc >= 0) & (loc < SEG) & (k != kn)
                    plsc.store_scatter(seg_v, [jnp.where(m, loc, 0)], v,
                                       mask=m)
                return c2

            lax.fori_loop(0, CHUNK // (L * UNROLL), vb, 0)
            return c

        lax.fori_loop(0, nch, cb, 0)
        pltpu.sync_copy(seg_v, out_hbm.at[pl.ds(base, SEG)])


def kernel(input, indices):
    idxf = indices.reshape(B, S_IN)
    rows = jnp.arange(B, dtype=jnp.int32)[:, None]
    gkey = (rows * S_OUT + idxf).reshape(-1)
    vals = input.reshape(-1)
    sk, sv = lax.sort((gkey, vals), dimension=0, is_stable=False, num_keys=1)
    skp = jnp.concatenate([sk, jnp.full((KBUF,), SENTINEL, jnp.int32)])
    svp = jnp.concatenate([sv, jnp.zeros((KBUF,), jnp.float32)])
    bounds = jnp.arange(NSEG + 1, dtype=jnp.int32) * SEG
    starts = jnp.searchsorted(
        sk, bounds, side="left", method="scan_unrolled").astype(jnp.int32)
    starts = jnp.concatenate(
        [starts, jnp.full((ST_LEN - NSEG - 1,), TOTAL, jnp.int32)])

    mesh = plsc.VectorSubcoreMesh(core_axis_name="c", subcore_axis_name="s")
    out = pl.kernel(
        _scatter_body,
        out_type=jax.ShapeDtypeStruct((OUT_TOTAL,), jnp.float32),
        scratch_types=[
            pltpu.VMEM((KBUF,), jnp.int32),
            pltpu.VMEM((CHUNK,), jnp.float32),
            pltpu.VMEM((L,), jnp.int32),
            pltpu.VMEM((SEG,), jnp.float32),
        ],
        mesh=mesh,
        compiler_params=pltpu.CompilerParams(needs_layout_passes=False),
    )(skp, svp, starts)
    return out.reshape(2, 32, 32, 112, 112)


# speedup vs baseline: 4.4871x; 1.0519x over previous
import jax
import jax.numpy as jnp
from jax import lax
from jax.experimental import pallas as pl
from jax.experimental.pallas import tpu as pltpu
from jax.experimental.pallas import tpu_sc as plsc

B = 64
S_IN = 50176
S_OUT = 401408
TOTAL = B * S_IN
OUT_TOTAL = B * S_OUT
NW = 32
SEG = 100352
NSEG = OUT_TOTAL // SEG
SEG_PER_W = NSEG // NW
CHUNK = 12544
KBUF = CHUNK + 16
UNROLL = 4
ZUNROLL = 8
SENTINEL = 0x7F000000
L = 16
BSTEPS = 22


def _scatter_body(sk_hbm, sv_hbm, out_hbm, key_v, val_v, gat_v, seg_v):
    wid = lax.axis_index("s") * 2 + lax.axis_index("c")
    lanes = lax.iota(jnp.int32, L)

    bvec = (wid * SEG_PER_W + lanes) * SEG

    def bs(_, lohi):
        lo, hi = lohi
        mid = (lo + hi) >> 1
        pltpu.sync_copy(sk_hbm.at[mid], gat_v)
        kmid = gat_v[...]
        pred = kmid < bvec
        return (jnp.where(pred, mid + 1, lo), jnp.where(pred, hi, mid))

    starts, _ = lax.fori_loop(
        0, BSTEPS, bs,
        (jnp.zeros((L,), jnp.int32), jnp.full((L,), TOTAL, jnp.int32)))

    for g in range(SEG_PER_W):
        base = (wid * SEG_PER_W + g) * SEG
        a0 = jnp.sum(jnp.where(lanes == g, starts, 0))
        a1 = jnp.sum(jnp.where(lanes == g + 1, starts, 0))
        off0 = (a0 // 8) * 8
        nch = (a1 - off0 + CHUNK - 1) // CHUNK

        def zb(i, c):
            for u in range(ZUNROLL):
                seg_v[pl.ds((i * ZUNROLL + u) * L, L)] = jnp.zeros(
                    (L,), jnp.float32)
            return c

        lax.fori_loop(0, SEG // (L * ZUNROLL), zb, 0)

        def cb(j, c):
            off = off0 + j * CHUNK
            pltpu.sync_copy(sk_hbm.at[pl.ds(off, KBUF)], key_v)
            pltpu.sync_copy(sv_hbm.at[pl.ds(off, CHUNK)], val_v)

            def vb(i, c2):
                for u in range(UNROLL):
                    b = (i * UNROLL + u) * L
                    k = key_v[pl.ds(b, L)]
                    kn = key_v[pl.ds(b + 1, L)]
                    v = val_v[pl.ds(b, L)]
                    loc = k - base
                    m = (loc >= 0) & (loc < SEG) & (k != kn)
                    plsc.store_scatter(seg_v, [jnp.where(m, loc, 0)], v,
                                       mask=m)
                return c2

            lax.fori_loop(0, CHUNK // (L * UNROLL), vb, 0)
            return c

        lax.fori_loop(0, nch, cb, 0)
        pltpu.sync_copy(seg_v, out_hbm.at[pl.ds(base, SEG)])


def kernel(input, indices):
    idxf = indices.reshape(B, S_IN)
    rows = jnp.arange(B, dtype=jnp.int32)[:, None]
    gkey = (rows * S_OUT + idxf).reshape(-1)
    vals = input.reshape(-1)
    sk, sv = lax.sort((gkey, vals), dimension=0, is_stable=False, num_keys=1)
    skp = jnp.concatenate([sk, jnp.full((KBUF,), SENTINEL, jnp.int32)])
    svp = jnp.concatenate([sv, jnp.zeros((KBUF,), jnp.float32)])

    mesh = plsc.VectorSubcoreMesh(core_axis_name="c", subcore_axis_name="s")
    out = pl.kernel(
        _scatter_body,
        out_type=jax.ShapeDtypeStruct((OUT_TOTAL,), jnp.float32),
        scratch_types=[
            pltpu.VMEM((KBUF,), jnp.int32),
            pltpu.VMEM((CHUNK,), jnp.float32),
            pltpu.VMEM((L,), jnp.int32),
            pltpu.VMEM((SEG,), jnp.float32),
        ],
        mesh=mesh,
        compiler_params=pltpu.CompilerParams(needs_layout_passes=False),
    )(skp, svp)
    return out.reshape(2, 32, 32, 112, 112)


# speedup vs baseline: 4.4997x; 1.0028x over previous
import jax
import jax.numpy as jnp
from jax import lax
from jax.experimental import pallas as pl
from jax.experimental.pallas import tpu as pltpu
from jax.experimental.pallas import tpu_sc as plsc

B = 64
S_IN = 50176
S_OUT = 401408
TOTAL = B * S_IN
OUT_TOTAL = B * S_OUT
NW = 32
SEG = 100352
NSEG = OUT_TOTAL // SEG
SEG_PER_W = NSEG // NW
CHUNK = 12544
KBUF = CHUNK + 16
UNROLL = 4
ZUNROLL = 8
L = 16
BSTEPS = 22


def _scatter_body(sk_hbm, sv_hbm, out_hbm, key_v, val_v, gat_v, seg_v):
    wid = lax.axis_index("s") * 2 + lax.axis_index("c")
    lanes = lax.iota(jnp.int32, L)

    bvec = (wid * SEG_PER_W + lanes) * SEG

    def bs(_, lohi):
        lo, hi = lohi
        mid = (lo + hi) >> 1
        pltpu.sync_copy(sk_hbm.at[mid], gat_v)
        kmid = gat_v[...]
        pred = kmid < bvec
        return (jnp.where(pred, mid + 1, lo), jnp.where(pred, hi, mid))

    starts, _ = lax.fori_loop(
        0, BSTEPS, bs,
        (jnp.zeros((L,), jnp.int32), jnp.full((L,), TOTAL, jnp.int32)))
    starts = jnp.minimum(starts, TOTAL)

    for g in range(SEG_PER_W):
        base = (wid * SEG_PER_W + g) * SEG
        a0 = jnp.sum(jnp.where(lanes == g, starts, 0))
        a1 = jnp.sum(jnp.where(lanes == g + 1, starts, 0))
        off0 = (a0 // 8) * 8
        nch = (a1 - off0 + CHUNK - 1) // CHUNK

        def zb(i, c):
            for u in range(ZUNROLL):
                seg_v[pl.ds((i * ZUNROLL + u) * L, L)] = jnp.zeros(
                    (L,), jnp.float32)
            return c

        lax.fori_loop(0, SEG // (L * ZUNROLL), zb, 0)

        def cb(j, c):
            off = off0 + j * CHUNK
            pltpu.sync_copy(sk_hbm.at[pl.ds(off, KBUF)], key_v)
            pltpu.sync_copy(sv_hbm.at[pl.ds(off, CHUNK)], val_v)

            def vb(i, c2):
                for u in range(UNROLL):
                    b = (i * UNROLL + u) * L
                    k = key_v[pl.ds(b, L)]
                    kn = key_v[pl.ds(b + 1, L)]
                    v = val_v[pl.ds(b, L)]
                    pos = off + b + lanes
                    loc = k - base
                    m = ((loc >= 0) & (loc < SEG)
                         & ((k != kn) | (pos == TOTAL - 1))
                         & (pos < TOTAL))
                    plsc.store_scatter(seg_v, [jnp.where(m, loc, 0)], v,
                                       mask=m)
                return c2

            lax.fori_loop(0, CHUNK // (L * UNROLL), vb, 0)
            return c

        lax.fori_loop(0, nch, cb, 0)
        pltpu.sync_copy(seg_v, out_hbm.at[pl.ds(base, SEG)])


def kernel(input, indices):
    idxf = indices.reshape(B, S_IN)
    rows = jnp.arange(B, dtype=jnp.int32)[:, None]
    gkey = (rows * S_OUT + idxf).reshape(-1)
    vals = input.reshape(-1)
    sk, sv = lax.sort((gkey, vals), dimension=0, is_stable=False, num_keys=1)

    mesh = plsc.VectorSubcoreMesh(core_axis_name="c", subcore_axis_name="s")
    out = pl.kernel(
        _scatter_body,
        out_type=jax.ShapeDtypeStruct((OUT_TOTAL,), jnp.float32),
        scratch_types=[
            pltpu.VMEM((KBUF,), jnp.int32),
            pltpu.VMEM((CHUNK,), jnp.float32),
            pltpu.VMEM((L,), jnp.int32),
            pltpu.VMEM((SEG,), jnp.float32),
        ],
        mesh=mesh,
        compiler_params=pltpu.CompilerParams(needs_layout_passes=False),
    )(sk, sv)
    return out.reshape(2, 32, 32, 112, 112)
